# Initial kernel scaffold; baseline (speedup 1.0000x reference)
import jax
import jax.numpy as jnp
from jax.experimental import pallas as pl


def kernel(days, day_table, W_weekend, b_weekend):
    raise NotImplementedError("write your pallas kernel here")


# speedup vs baseline: 1.1144x; 1.1144x over previous
import functools

import jax
import jax.numpy as jnp
from jax import lax
from jax.experimental import pallas as pl
from jax.experimental.pallas import tpu as pltpu
from jax.experimental.pallas import tpu_sc as plsc

EMBED_DIM = 32
HALF = 16

_NC = 2
_NS = 16
_NW = _NC * _NS

_CHUNK = 1024
_IDX_ROWS = _CHUNK // 128


def _fused_table(dt8, w8, b8):
    def body(dt_ref, w_ref, b_ref, out_ref):
        wk = (lax.broadcasted_iota(jnp.int32, (8, HALF), 0) >= 5).astype(
            jnp.float32)
        out_ref[...] = jnp.concatenate(
            [dt_ref[...], wk * w_ref[...] + b_ref[...]], axis=-1)

    return pl.pallas_call(
        body,
        out_shape=jax.ShapeDtypeStruct((8, EMBED_DIM), jnp.float32),
    )(dt8, w8, b8)


def _sc_gather(idx2d, table, n_total):
    per_w = n_total // _NW
    n_chunks = per_w // _CHUNK
    rows_per_w = per_w // 128

    mesh = plsc.VectorSubcoreMesh(
        core_axis_name="c", subcore_axis_name="s",
        num_cores=_NC, num_subcores=_NS)

    @functools.partial(
        pl.kernel,
        out_type=jax.ShapeDtypeStruct((n_total, EMBED_DIM), jnp.float32),
        mesh=mesh,
        scratch_types=[
            pltpu.VMEM((_IDX_ROWS, 128), jnp.int32),
            pltpu.VMEM((_CHUNK, EMBED_DIM), jnp.float32),
            pltpu.SemaphoreType.DMA,
        ],
        compiler_params=pltpu.CompilerParams(use_tc_tiling_on_sc=False),
    )
    def k(idx_hbm, table_hbm, out_hbm, idx_v, rows_v, sem):
        wid = lax.axis_index("s") * _NC + lax.axis_index("c")
        row_base = wid * rows_per_w
        out_base = wid * per_w

        def chunk_body(ck, carry):
            pltpu.sync_copy(
                idx_hbm.at[pl.ds(row_base + ck * _IDX_ROWS, _IDX_ROWS)], idx_v)
            cps = [
                pltpu.async_copy(
                    table_hbm.at[idx_v.at[j]],
                    rows_v.at[pl.ds(j * 128, 128)],
                    sem)
                for j in range(_IDX_ROWS)
            ]
            for cp in cps:
                cp.wait()
            pltpu.sync_copy(
                rows_v, out_hbm.at[pl.ds(out_base + ck * _CHUNK, _CHUNK)])
            return carry

        lax.fori_loop(0, n_chunks, chunk_body, 0)

    return k(idx2d, table)


def kernel(days, day_table, W_weekend, b_weekend):
    shape = days.shape
    n_total = days.size
    idx2d = days.reshape(n_total // 128, 128)
    dt8 = jnp.zeros((8, HALF), jnp.float32).at[:7].set(day_table)
    w8 = jnp.broadcast_to(W_weekend.reshape(1, HALF), (8, HALF))
    b8 = jnp.broadcast_to(b_weekend.reshape(1, HALF), (8, HALF))
    table = _fused_table(dt8, w8, b8)
    out = _sc_gather(idx2d, table, n_total)
    return out.reshape(*shape, EMBED_DIM)


# speedup vs baseline: 7.2566x; 6.5117x over previous
import functools

import jax
import jax.numpy as jnp
from jax import lax
from jax.experimental import pallas as pl
from jax.experimental.pallas import tpu as pltpu
from jax.experimental.pallas import tpu_sc as plsc

EMBED_DIM = 32
HALF = 16
LANES = 16

_NC = 2
_NS = 16
_NW = _NC * _NS

_CHUNK = 1600
_GROUPS = _CHUNK // LANES


def _vperm(src, idx):
    return lax.gather(
        src, idx[:, None],
        dimension_numbers=lax.GatherDimensionNumbers(
            offset_dims=(), collapsed_slice_dims=(0,), start_index_map=(0,)),
        slice_sizes=(1,),
        mode=lax.GatherScatterMode.PROMISE_IN_BOUNDS)


def _sc_lookup(days_flat, dtT, w_vec, b_vec, n_total):
    per_w = n_total // _NW
    n_chunks = per_w // _CHUNK
    n_half = n_chunks // 2

    mesh = plsc.VectorSubcoreMesh(
        core_axis_name="c", subcore_axis_name="s",
        num_cores=_NC, num_subcores=_NS)

    @functools.partial(
        pl.kernel,
        out_type=jax.ShapeDtypeStruct((n_total * EMBED_DIM,), jnp.float32),
        mesh=mesh,
        scratch_types=[
            pltpu.VMEM((_CHUNK,), jnp.int32),
            pltpu.VMEM((_CHUNK,), jnp.int32),
            pltpu.VMEM((_CHUNK * EMBED_DIM,), jnp.float32),
            pltpu.VMEM((_CHUNK * EMBED_DIM,), jnp.float32),
            pltpu.VMEM((LANES, LANES), jnp.float32),
            pltpu.VMEM((LANES,), jnp.float32),
            pltpu.VMEM((LANES,), jnp.float32),
            pltpu.SemaphoreType.DMA,
            pltpu.SemaphoreType.DMA,
            pltpu.SemaphoreType.DMA,
            pltpu.SemaphoreType.DMA,
        ],
        compiler_params=pltpu.CompilerParams(needs_layout_passes=False),
    )
    def k(idx_hbm, dtT_hbm, w_hbm, b_hbm, out_hbm,
          idx_v0, idx_v1, out_v0, out_v1, dtT_v, w_v, b_v,
          sem_in0, sem_in1, sem_out0, sem_out1):
        idx_v = (idx_v0, idx_v1)
        out_v = (out_v0, out_v1)
        sem_in = (sem_in0, sem_in1)
        sem_out = (sem_out0, sem_out1)
        wid = lax.axis_index("s") * _NC + lax.axis_index("c")
        in_base = wid * per_w

        pltpu.sync_copy(dtT_hbm, dtT_v)
        pltpu.sync_copy(w_hbm, w_v)
        pltpu.sync_copy(b_hbm, b_v)
        w_all = w_v[...]
        b_all = b_v[...]
        wk = jnp.where(lax.iota(jnp.int32, LANES) >= 5, 1.0, 0.0)
        cols = [dtT_v[j] for j in range(HALF)]
        for j in range(HALF):
            sel = jnp.full((LANES,), j, jnp.int32)
            cols.append(wk * _vperm(w_all, sel) + _vperm(b_all, sel))
        pos0 = lax.iota(jnp.int32, LANES) * EMBED_DIM

        def in_cp(ck, b):
            off = pl.multiple_of(in_base + ck * _CHUNK, _CHUNK)
            return pltpu.make_async_copy(
                idx_hbm.at[pl.ds(off, _CHUNK)], idx_v[b], sem_in[b])

        def out_cp(ck, b):
            off = pl.multiple_of(
                (in_base + ck * _CHUNK) * EMBED_DIM, _CHUNK * EMBED_DIM)
            return pltpu.make_async_copy(
                out_v[b], out_hbm.at[pl.ds(off, _CHUNK * EMBED_DIM)],
                sem_out[b])

        in_cp(0, 0).start()
        in_cp(1, 1).start()

        def chunk_pair(K, carry):
            for b in range(2):
                ck = 2 * K + b
                in_cp(ck, b).wait()

                @pl.when(K >= 1)
                def _():
                    out_cp(ck - 2, b).wait()

                idx_ref = idx_v[b]
                out_ref = out_v[b]

                def group(g, carry2):
                    start = pl.multiple_of(g * LANES, LANES)
                    dvec = idx_ref[pl.ds(start, LANES)]
                    pos = pos0 + g * (LANES * EMBED_DIM)
                    for j in range(EMBED_DIM):
                        plsc.store_scatter(
                            out_ref, [pos + j], _vperm(cols[j], dvec))
                    return carry2

                lax.fori_loop(0, _GROUPS, group, 0)
                out_cp(ck, b).start()

                @pl.when(K < n_half - 1)
                def _():
                    in_cp(ck + 2, b).start()
            return carry

        lax.fori_loop(0, n_half, chunk_pair, 0)
        out_cp(n_chunks - 2, 0).wait()
        out_cp(n_chunks - 1, 1).wait()

    return k(days_flat, dtT, w_vec, b_vec)


def kernel(days, day_table, W_weekend, b_weekend):
    shape = days.shape
    n_total = days.size
    days_flat = days.reshape(n_total)
    dtT = jnp.zeros((LANES, LANES), jnp.float32).at[:, :7].set(day_table.T)
    out = _sc_lookup(days_flat, dtT, W_weekend.reshape(HALF),
                     b_weekend.reshape(HALF), n_total)
    return out.reshape(*shape, EMBED_DIM)


# speedup vs baseline: 82.9484x; 11.4307x over previous
import functools

import jax
import jax.numpy as jnp
from jax import lax
from jax.experimental import pallas as pl
from jax.experimental.pallas import tpu as pltpu
from jax.experimental.pallas import tpu_sc as plsc

EMBED_DIM = 32
HALF = 16
LANES = 16

_NC = 2
_NS = 16
_NW = _NC * _NS

_T = 200
_I = 16384
_JB = 8
_CI = 4096
_NCH = _I // _CI
_UNITS = _T * (EMBED_DIM // _JB)
_UPW = _UNITS // _NW
_STEPS = _UPW * _NCH
_GROUPS = _CI // LANES


def _vperm(src, idx):
    return lax.gather(
        src, idx[:, None],
        dimension_numbers=lax.GatherDimensionNumbers(
            offset_dims=(), collapsed_slice_dims=(0,), start_index_map=(0,)),
        slice_sizes=(1,),
        mode=lax.GatherScatterMode.PROMISE_IN_BOUNDS)


def _sc_lookup(daysT_flat, dtT_flat, w_vec, b_vec):
    mesh = plsc.VectorSubcoreMesh(
        core_axis_name="c", subcore_axis_name="s",
        num_cores=_NC, num_subcores=_NS)

    @functools.partial(
        pl.kernel,
        out_type=jax.ShapeDtypeStruct((_T, EMBED_DIM, _I), jnp.float32),
        mesh=mesh,
        scratch_types=[
            pltpu.VMEM((_CI,), jnp.int32),
            pltpu.VMEM((_CI,), jnp.int32),
            pltpu.VMEM((_JB, _CI), jnp.float32),
            pltpu.VMEM((_JB, _CI), jnp.float32),
            pltpu.VMEM((EMBED_DIM * LANES,), jnp.float32),
            pltpu.VMEM((LANES,), jnp.float32),
            pltpu.VMEM((LANES,), jnp.float32),
            pltpu.SemaphoreType.DMA,
            pltpu.SemaphoreType.DMA,
            pltpu.SemaphoreType.DMA,
            pltpu.SemaphoreType.DMA,
        ],
        compiler_params=pltpu.CompilerParams(needs_layout_passes=False),
    )
    def k(idx_hbm, tbl_hbm, w_hbm, b_hbm, out_hbm,
          idx_v0, idx_v1, out_v0, out_v1, fused_v, w_v, b_v,
          sem_in0, sem_in1, sem_out0, sem_out1):
        idx_v = (idx_v0, idx_v1)
        out_v = (out_v0, out_v1)
        sem_in = (sem_in0, sem_in1)
        sem_out = (sem_out0, sem_out1)
        wid = lax.axis_index("s") * _NC + lax.axis_index("c")
        u_base = wid * _UPW

        pltpu.sync_copy(tbl_hbm, fused_v.at[pl.ds(0, HALF * LANES)])
        pltpu.sync_copy(w_hbm, w_v)
        pltpu.sync_copy(b_hbm, b_v)
        w_all = w_v[...]
        b_all = b_v[...]
        wk = jnp.where(lax.iota(jnp.int32, LANES) >= 5, 1.0, 0.0)
        for j in range(HALF):
            sel = jnp.full((LANES,), j, jnp.int32)
            fused_v[pl.ds((HALF + j) * LANES, LANES)] = (
                wk * _vperm(w_all, sel) + _vperm(b_all, sel))

        def unit_of(s):
            u = u_base + (s >> 2)
            return u >> 2, u & 3, s & 3

        def in_cp(s, b):
            t, _, ch = unit_of(s)
            off = pl.multiple_of(t * _I + ch * _CI, _CI)
            return pltpu.make_async_copy(
                idx_hbm.at[pl.ds(off, _CI)], idx_v[b], sem_in[b])

        def out_cp(s, b):
            t, jb, ch = unit_of(s)
            return pltpu.make_async_copy(
                out_v[b],
                out_hbm.at[t, pl.ds(jb * _JB, _JB),
                           pl.ds(pl.multiple_of(ch * _CI, _CI), _CI)],
                sem_out[b])

        in_cp(0, 0).start()
        in_cp(1, 1).start()

        def step_pair(K, carry):
            for b in range(2):
                s = 2 * K + b
                in_cp(s, b).wait()

                @pl.when(K >= 1)
                def _():
                    out_cp(s - 2, b).wait()

                _, jb, _ = unit_of(s)
                cbase = pl.multiple_of(jb * (_JB * LANES), _JB * LANES)
                colv = [fused_v[pl.ds(cbase + jj * LANES, LANES)]
                        for jj in range(_JB)]
                idx_ref = idx_v[b]
                out_ref = out_v[b]

                def group(g, carry2):
                    span = pl.ds(pl.multiple_of(g * LANES, LANES), LANES)
                    dvec = idx_ref[span]
                    for jj in range(_JB):
                        out_ref[jj, span] = _vperm(colv[jj], dvec)
                    return carry2

                lax.fori_loop(0, _GROUPS, group, 0)
                out_cp(s, b).start()

                @pl.when(K < _STEPS // 2 - 1)
                def _():
                    in_cp(s + 2, b).start()
            return carry

        lax.fori_loop(0, _STEPS // 2, step_pair, 0)
        out_cp(_STEPS - 2, 0).wait()
        out_cp(_STEPS - 1, 1).wait()

    return k(daysT_flat, dtT_flat, w_vec, b_vec)


def kernel(days, day_table, W_weekend, b_weekend):
    daysT_flat = days.T.reshape(_T * _I)
    dtT_flat = jnp.zeros((LANES, LANES), jnp.float32).at[:, :7].set(
        day_table.T).reshape(HALF * LANES)
    a = _sc_lookup(daysT_flat, dtT_flat, W_weekend.reshape(HALF),
                   b_weekend.reshape(HALF))
    return a.transpose(2, 0, 1)
